# baseline (speedup 1.0000x reference)
import functools

import jax
import jax.numpy as jnp
from jax import lax
from jax.experimental import pallas as pl
from jax.experimental.pallas import tpu as pltpu
from jax.experimental.pallas import tpu_sc as plsc

_NUM = 8192
_RATE = 0.005
_LANES = 16


def _sample_indices(n):
    probs = _RATE * jnp.exp(-_RATE * jnp.arange(n, dtype=jnp.float32))
    gumbel = jax.random.gumbel(jax.random.key(42), (n,), dtype=jnp.float32)
    logits = jnp.log(probs)
    _, ind = jax.lax.top_k(logits + gumbel, _NUM)
    return ind


@functools.cache
def _constant_indices(n):
    with jax.ensure_compile_time_eval():
        ind = jnp.asarray(_sample_indices(n), jnp.int32)
        kmax = int(jnp.max(ind)) + 1
    return ind, kmax


def _make_gather_kernel(batch, ncols, kp):
    info = plsc.get_sparse_core_info()
    nc, ns = info.num_cores, info.num_subcores
    nw = nc * ns
    rows_per_w = batch // nw
    chunks = _NUM // _LANES
    mesh = plsc.VectorSubcoreMesh(core_axis_name="c", subcore_axis_name="s")

    @functools.partial(
        pl.kernel,
        out_type=jax.ShapeDtypeStruct((batch * _NUM,), jnp.float32),
        mesh=mesh,
        scratch_types=[
            pltpu.VMEM((_NUM,), jnp.int32),
            pltpu.VMEM((kp,), jnp.float32),
            pltpu.VMEM((_NUM,), jnp.float32),
        ],
        compiler_params=pltpu.CompilerParams(needs_layout_passes=False),
    )
    def gather_rows(samples_hbm, ind_hbm, out_hbm, idx_v, row_buf, out_buf):
        wid = lax.axis_index("s") * nc + lax.axis_index("c")
        base = wid * rows_per_w
        pltpu.sync_copy(ind_hbm, idx_v)
        for r in range(rows_per_w):
            row = base + r
            pltpu.sync_copy(samples_hbm.at[pl.ds(row * ncols, kp)], row_buf)

            def inner(c, _):
                iv = idx_v[pl.ds(c * _LANES, _LANES)]
                out_buf[pl.ds(c * _LANES, _LANES)] = plsc.load_gather(
                    row_buf, [iv]
                )
                return _

            lax.fori_loop(0, chunks, inner, 0)
            pltpu.sync_copy(out_buf, out_hbm.at[pl.ds(row * _NUM, _NUM)])

    return gather_rows


def kernel(samples):
    batch, ncols = samples.shape
    ind, kmax = _constant_indices(ncols)
    kp = min(-(-kmax // _LANES) * _LANES, ncols)
    gather = _make_gather_kernel(batch, ncols, kp)
    out = gather(samples.reshape(-1), ind)
    return out.reshape(batch, _NUM)
